# baseline (speedup 1.0000x reference)
import functools

import jax
import jax.numpy as jnp
from jax import lax
from jax.experimental import pallas as pl
from jax.experimental.pallas import tpu as pltpu
from jax.experimental.pallas import tpu_sc as plsc

NC = 2
NS = 16
NW = NC * NS
CH = 80


def _vmesh():
    return plsc.VectorSubcoreMesh(
        core_axis_name="c", subcore_axis_name="s",
        num_cores=NC, num_subcores=NS)


def _pre_body(x_ref, ws_ref, wt_ref, xs_ref, xt_ref):
    x = x_ref[...]
    xs_ref[...] = jnp.dot(x, ws_ref[...], preferred_element_type=jnp.float32)
    xt_ref[...] = jnp.dot(x, wt_ref[...], preferred_element_type=jnp.float32)


def _tc_precompute(x, Ws, Wt, bn):
    N, D = x.shape
    H = Ws.shape[1]
    grid = (N // bn,)
    return pl.pallas_call(
        _pre_body,
        grid=grid,
        in_specs=[pl.BlockSpec((bn, D), lambda i: (i, 0)),
                  pl.BlockSpec((D, H), lambda i: (0, 0)),
                  pl.BlockSpec((D, H), lambda i: (0, 0))],
        out_specs=[pl.BlockSpec((bn, H), lambda i: (i, 0)),
                   pl.BlockSpec((bn, H), lambda i: (i, 0))],
        out_shape=[jax.ShapeDtypeStruct((N, H), jnp.float32),
                   jax.ShapeDtypeStruct((N, H), jnp.float32)],
    )(x, Ws, Wt)


def _edge_body(gs_ref, gt_ref, attr_ref, mask_ref, wa_ref, be1_ref,
               we2_ref, be2_ref, ef_ref):
    pre = (gs_ref[...] + gt_ref[...]
           + jnp.dot(attr_ref[...], wa_ref[...],
                     preferred_element_type=jnp.float32)
           + be1_ref[...])
    h = jnp.maximum(pre, 0.0)
    ef_ref[...] = (jnp.dot(h, we2_ref[...],
                           preferred_element_type=jnp.float32)
                   + be2_ref[...]) * mask_ref[...]


def _edge_body_alias(ef_in_ref, gs_ref, gt_ref, attr_ref, mask_ref, wa_ref,
                     be1_ref, we2_ref, be2_ref, ef_ref):
    _edge_body(gs_ref, gt_ref, attr_ref, mask_ref, wa_ref, be1_ref,
               we2_ref, be2_ref, ef_ref)


def _tc_edge(gs, gt, attr, mask, Wa, be1, We2, be2, be, Etot, blk0, ef_prev):
    Eh, H = gs.shape
    DE = attr.shape[1]
    grid = (Eh // be,)
    in_specs = [pl.BlockSpec((be, H), lambda i: (i, 0)),
                pl.BlockSpec((be, H), lambda i: (i, 0)),
                pl.BlockSpec((be, DE), lambda i: (i, 0)),
                pl.BlockSpec((be, 1), lambda i: (i, 0)),
                pl.BlockSpec((DE, H), lambda i: (0, 0)),
                pl.BlockSpec((1, H), lambda i: (0, 0)),
                pl.BlockSpec((H, H), lambda i: (0, 0)),
                pl.BlockSpec((1, H), lambda i: (0, 0))]
    out_specs = pl.BlockSpec((be, H), lambda i: (i + blk0, 0))
    if ef_prev is None:
        return pl.pallas_call(
            _edge_body,
            grid=grid,
            in_specs=in_specs,
            out_specs=out_specs,
            out_shape=jax.ShapeDtypeStruct((Etot, H), jnp.float32),
        )(gs, gt, attr, mask, Wa, be1, We2, be2)
    return pl.pallas_call(
        _edge_body_alias,
        grid=grid,
        in_specs=[pl.BlockSpec(memory_space=pl.ANY)] + in_specs,
        out_specs=out_specs,
        out_shape=jax.ShapeDtypeStruct((Etot, H), jnp.float32),
        input_output_aliases={0: 0},
    )(ef_prev, gs, gt, attr, mask, Wa, be1, We2, be2)


def _node_body(x_ref, p0_ref, p1_ref, w1x_ref, w1a_ref, bn1_ref,
               w2_ref, bn2_ref, o_ref):
    agg = p0_ref[...] + p1_ref[...]
    h = jnp.maximum(
        jnp.dot(x_ref[...], w1x_ref[...], preferred_element_type=jnp.float32)
        + jnp.dot(agg, w1a_ref[...], preferred_element_type=jnp.float32)
        + bn1_ref[...], 0.0)
    o_ref[...] = (jnp.dot(h, w2_ref[...], preferred_element_type=jnp.float32)
                  + bn2_ref[...])


def _tc_node(x, p0, p1, W1x, W1a, bn1, W2, bn2, bn):
    N, D = x.shape
    H = W1x.shape[1]
    Do = W2.shape[1]
    grid = (N // bn,)
    return pl.pallas_call(
        _node_body,
        grid=grid,
        in_specs=[pl.BlockSpec((bn, D), lambda i: (i, 0)),
                  pl.BlockSpec((bn, H), lambda i: (i, 0)),
                  pl.BlockSpec((bn, H), lambda i: (i, 0)),
                  pl.BlockSpec((D, H), lambda i: (0, 0)),
                  pl.BlockSpec((H, H), lambda i: (0, 0)),
                  pl.BlockSpec((1, H), lambda i: (0, 0)),
                  pl.BlockSpec((H, Do), lambda i: (0, 0)),
                  pl.BlockSpec((1, Do), lambda i: (0, 0))],
        out_specs=pl.BlockSpec((bn, Do), lambda i: (i, 0)),
        out_shape=jax.ShapeDtypeStruct((N, Do), jnp.float32),
    )(x, p0, p1, W1x, W1a, bn1, W2, bn2)


NBUF = 5


def _sc_gather(xs, xt, row, col, CH):
    N, H = xs.shape
    E = row.shape[0]
    epw = E // NW
    nch = epw // CH
    ngrp = nch // NBUF

    @functools.partial(
        pl.kernel,
        out_type=[jax.ShapeDtypeStruct((E, H), jnp.float32),
                  jax.ShapeDtypeStruct((E, H), jnp.float32)],
        mesh=_vmesh(),
        scratch_types=(
            [pltpu.VMEM((epw,), jnp.int32)] * 2
            + [pltpu.VMEM((CH, H), jnp.float32)] * (2 * NBUF)
            + [pltpu.SemaphoreType.DMA] * (2 * NBUF)
        ),
    )
    def k(xs_hbm, xt_hbm, row_hbm, col_hbm, gs_hbm, gt_hbm, *scr):
        idx_r, idx_c = scr[0], scr[1]
        rs = scr[2:2 + NBUF]
        rt = scr[2 + NBUF:2 + 2 * NBUF]
        sem_g = scr[2 + 2 * NBUF:2 + 3 * NBUF]
        sem_w = scr[2 + 3 * NBUF:2 + 4 * NBUF]
        wid = lax.axis_index("s") * NC + lax.axis_index("c")
        base = wid * epw
        pltpu.sync_copy(row_hbm.at[pl.ds(base, epw)], idx_r)
        pltpu.sync_copy(col_hbm.at[pl.ds(base, epw)], idx_c)

        def gstart(b, off):
            pltpu.make_async_copy(
                xs_hbm.at[idx_r.at[pl.ds(off, CH)]], rs[b], sem_g[b]).start()
            pltpu.make_async_copy(
                xt_hbm.at[idx_c.at[pl.ds(off, CH)]], rt[b], sem_g[b]).start()

        def gwait(b, off):
            pltpu.make_async_copy(
                xs_hbm.at[idx_r.at[pl.ds(off, CH)]], rs[b], sem_g[b]).wait()
            pltpu.make_async_copy(
                xt_hbm.at[idx_c.at[pl.ds(off, CH)]], rt[b], sem_g[b]).wait()

        def wstart(b, off):
            pltpu.make_async_copy(
                rs[b], gs_hbm.at[pl.ds(base + off, CH)], sem_w[b]).start()
            pltpu.make_async_copy(
                rt[b], gt_hbm.at[pl.ds(base + off, CH)], sem_w[b]).start()

        def wwait(b, off):
            pltpu.make_async_copy(
                rs[b], gs_hbm.at[pl.ds(base + off, CH)], sem_w[b]).wait()
            pltpu.make_async_copy(
                rt[b], gt_hbm.at[pl.ds(base + off, CH)], sem_w[b]).wait()

        for b in range(NBUF):
            gstart(b, b * CH)

        def group(g, carry):
            for b in range(NBUF):
                off = (g * NBUF + b) * CH
                gwait(b, off)
                wstart(b, off)
                wwait(b, off)
                gstart(b, off + NBUF * CH)
            return carry

        lax.fori_loop(0, ngrp - 1, group, 0)
        for b in range(NBUF):
            off = ((ngrp - 1) * NBUF + b) * CH
            gwait(b, off)
            wstart(b, off)
            wwait(b, off)

    return k(xs, xt, row, col)


def _sc_scatter(ef, row, zrows, Np):
    E, H = ef.shape
    CHS = 40
    epw = E // NW
    nch = epw // CHS
    rpt = Np // NS
    zb = zrows.shape[0]

    ngrp = nch // NBUF

    @functools.partial(
        pl.kernel,
        out_type=jax.ShapeDtypeStruct((NC, Np, H), jnp.float32),
        mesh=_vmesh(),
        scratch_types=(
            [pltpu.VMEM_SHARED((Np, H), jnp.float32)]
            + [pltpu.VMEM((CHS, H), jnp.float32)] * NBUF
            + [pltpu.VMEM((CHS,), jnp.int32)] * NBUF
            + [pltpu.VMEM((zb, H), jnp.float32)]
            + [pltpu.SemaphoreType.DMA] * (2 * NBUF + 1)
        ),
    )
    def k(ef_hbm, row_hbm, z_hbm, out_hbm, *scr):
        acc = scr[0]
        efb = scr[1:1 + NBUF]
        idxb = scr[1 + NBUF:1 + 2 * NBUF]
        zbuf = scr[1 + 2 * NBUF]
        sem_l = scr[2 + 2 * NBUF:2 + 3 * NBUF]
        sem_a = scr[2 + 3 * NBUF:2 + 4 * NBUF]
        sem_z = scr[2 + 4 * NBUF]
        cid = lax.axis_index("c")
        sid = lax.axis_index("s")
        wid = sid * NC + cid
        base = wid * epw
        pltpu.sync_copy(z_hbm, zbuf)
        for j in range(rpt // zb):
            pltpu.make_async_copy(
                zbuf, acc.at[pl.ds(sid * rpt + j * zb, zb)], sem_z).start()
        for j in range(rpt // zb):
            pltpu.make_async_copy(
                zbuf, acc.at[pl.ds(sid * rpt + j * zb, zb)], sem_z).wait()
        plsc.subcore_barrier()

        def lstart(b, off):
            pltpu.make_async_copy(
                row_hbm.at[pl.ds(base + off, CHS)], idxb[b], sem_l[b]).start()
            pltpu.make_async_copy(
                ef_hbm.at[pl.ds(base + off, CHS)], efb[b], sem_l[b]).start()

        def lwait(b, off):
            pltpu.make_async_copy(
                row_hbm.at[pl.ds(base + off, CHS)], idxb[b], sem_l[b]).wait()
            pltpu.make_async_copy(
                ef_hbm.at[pl.ds(base + off, CHS)], efb[b], sem_l[b]).wait()

        for b in range(NBUF):
            lstart(b, b * CHS)

        def group(g, carry):
            for b in range(NBUF):
                off = (g * NBUF + b) * CHS
                lwait(b, off)
                pltpu.sync_copy(efb[b], acc.at[idxb[b]], add=True)
                lstart(b, off + NBUF * CHS)
            return carry

        lax.fori_loop(0, ngrp - 1, group, 0)
        for b in range(NBUF):
            off = ((ngrp - 1) * NBUF + b) * CHS
            lwait(b, off)
            pltpu.sync_copy(efb[b], acc.at[idxb[b]], add=True)
        plsc.subcore_barrier()

        nfl = rpt // CHS
        for j in range(nfl):
            b = j % NBUF
            r0 = sid * rpt + j * CHS
            if j >= NBUF:
                pltpu.make_async_copy(
                    efb[b], out_hbm.at[cid, pl.ds(sid * rpt + (j - NBUF) * CHS, CHS)],
                    sem_a[b]).wait()
            pltpu.sync_copy(acc.at[pl.ds(r0, CHS)], efb[b])
            pltpu.make_async_copy(
                efb[b], out_hbm.at[cid, pl.ds(r0, CHS)], sem_a[b]).start()
        for j in range(max(nfl - NBUF, 0), nfl):
            b = j % NBUF
            pltpu.make_async_copy(
                efb[b], out_hbm.at[cid, pl.ds(sid * rpt + j * CHS, CHS)],
                sem_a[b]).wait()

    return k(ef, row, zrows)


def kernel(x, edge_index, edge_mask, edge_attr,
           We1, be1, We2, be2, Wn1, bn1, Wn2, bn2):
    N, D = x.shape
    E = edge_index.shape[1]
    E2 = E // 2
    H = We2.shape[0]
    Do = Wn2.shape[1]
    row = edge_index[0]
    col = edge_index[1]
    Ws, Wt, Wa = We1[:D], We1[D:2 * D], We1[2 * D:]
    b1 = be1.reshape(1, H)
    b2 = be2.reshape(1, H)

    xs, xt = _tc_precompute(x, Ws, Wt, bn=1000)
    gsA, gtA = _sc_gather(xs, xt, row[:E2], col[:E2], CH=40)
    rowB, colB, gsA, gtA = lax.optimization_barrier(
        (row[E2:], col[E2:], gsA, gtA))
    gsB, gtB = _sc_gather(xs, xt, rowB, colB, CH=40)
    efA = _tc_edge(gsA, gtA, edge_attr[:E2], edge_mask[:E2], Wa, b1, We2, b2,
                   be=2000, Etot=E, blk0=0, ef_prev=None)
    ef = _tc_edge(gsB, gtB, edge_attr[E2:], edge_mask[E2:], Wa, b1, We2, b2,
                  be=2000, Etot=E, blk0=E2 // 2000, ef_prev=efA)
    Np = ((N + 128 * NS - 1) // (128 * NS)) * (128 * NS)
    z = jnp.zeros((128, H), jnp.float32)
    parts = _sc_scatter(ef, row, z, Np)
    x_out = _tc_node(x, parts[0], parts[1], Wn1[:D], Wn1[D:],
                     bn1.reshape(1, H), Wn2, bn2.reshape(1, Do), bn=1000)
    x_out, *_ = lax.optimization_barrier(
        (x_out, row, col, xs, xt, gsA, gtA, gsB, gtB, z, parts))
    return (x_out, ef)


# speedup vs baseline: 1.1782x; 1.1782x over previous
import functools

import jax
import jax.numpy as jnp
from jax import lax
from jax.experimental import pallas as pl
from jax.experimental.pallas import tpu as pltpu
from jax.experimental.pallas import tpu_sc as plsc

NC = 2
NS = 16
NW = NC * NS
CH = 80


def _vmesh():
    return plsc.VectorSubcoreMesh(
        core_axis_name="c", subcore_axis_name="s",
        num_cores=NC, num_subcores=NS)


def _pre_body(x_ref, ws_ref, wt_ref, xs_ref, xt_ref):
    x = x_ref[...]
    xs_ref[...] = jnp.dot(x, ws_ref[...], preferred_element_type=jnp.float32)
    xt_ref[...] = jnp.dot(x, wt_ref[...], preferred_element_type=jnp.float32)


def _tc_precompute(x, Ws, Wt, bn):
    N, D = x.shape
    H = Ws.shape[1]
    grid = (N // bn,)
    return pl.pallas_call(
        _pre_body,
        grid=grid,
        in_specs=[pl.BlockSpec((bn, D), lambda i: (i, 0)),
                  pl.BlockSpec((D, H), lambda i: (0, 0)),
                  pl.BlockSpec((D, H), lambda i: (0, 0))],
        out_specs=[pl.BlockSpec((bn, H), lambda i: (i, 0)),
                   pl.BlockSpec((bn, H), lambda i: (i, 0))],
        out_shape=[jax.ShapeDtypeStruct((N, H), jnp.float32),
                   jax.ShapeDtypeStruct((N, H), jnp.float32)],
    )(x, Ws, Wt)


def _edge_body(pre_ref, attr_ref, mask_ref, wa_ref, be1_ref,
               we2_ref, be2_ref, ef_ref):
    pre = (pre_ref[...]
           + jnp.dot(attr_ref[...], wa_ref[...],
                     preferred_element_type=jnp.float32)
           + be1_ref[...])
    h = jnp.maximum(pre, 0.0)
    ef_ref[...] = (jnp.dot(h, we2_ref[...],
                           preferred_element_type=jnp.float32)
                   + be2_ref[...]) * mask_ref[...]


def _tc_edge(pre, attr, mask, Wa, be1, We2, be2, be):
    E, H = pre.shape
    DE = attr.shape[1]
    grid = (E // be,)
    return pl.pallas_call(
        _edge_body,
        grid=grid,
        in_specs=[pl.BlockSpec((be, H), lambda i: (i, 0)),
                  pl.BlockSpec((be, DE), lambda i: (i, 0)),
                  pl.BlockSpec((be, 1), lambda i: (i, 0)),
                  pl.BlockSpec((DE, H), lambda i: (0, 0)),
                  pl.BlockSpec((1, H), lambda i: (0, 0)),
                  pl.BlockSpec((H, H), lambda i: (0, 0)),
                  pl.BlockSpec((1, H), lambda i: (0, 0))],
        out_specs=pl.BlockSpec((be, H), lambda i: (i, 0)),
        out_shape=jax.ShapeDtypeStruct((E, H), jnp.float32),
    )(pre, attr, mask, Wa, be1, We2, be2)


def _node_body(x_ref, p0_ref, p1_ref, w1x_ref, w1a_ref, bn1_ref,
               w2_ref, bn2_ref, o_ref):
    agg = p0_ref[...] + p1_ref[...]
    h = jnp.maximum(
        jnp.dot(x_ref[...], w1x_ref[...], preferred_element_type=jnp.float32)
        + jnp.dot(agg, w1a_ref[...], preferred_element_type=jnp.float32)
        + bn1_ref[...], 0.0)
    o_ref[...] = (jnp.dot(h, w2_ref[...], preferred_element_type=jnp.float32)
                  + bn2_ref[...])


def _tc_node(x, p0, p1, W1x, W1a, bn1, W2, bn2, bn):
    N, D = x.shape
    H = W1x.shape[1]
    Do = W2.shape[1]
    grid = (N // bn,)
    return pl.pallas_call(
        _node_body,
        grid=grid,
        in_specs=[pl.BlockSpec((bn, D), lambda i: (i, 0)),
                  pl.BlockSpec((bn, H), lambda i: (i, 0)),
                  pl.BlockSpec((bn, H), lambda i: (i, 0)),
                  pl.BlockSpec((D, H), lambda i: (0, 0)),
                  pl.BlockSpec((H, H), lambda i: (0, 0)),
                  pl.BlockSpec((1, H), lambda i: (0, 0)),
                  pl.BlockSpec((H, Do), lambda i: (0, 0)),
                  pl.BlockSpec((1, Do), lambda i: (0, 0))],
        out_specs=pl.BlockSpec((bn, Do), lambda i: (i, 0)),
        out_shape=jax.ShapeDtypeStruct((N, Do), jnp.float32),
    )(x, p0, p1, W1x, W1a, bn1, W2, bn2)


NBUF = 5


def _sc_gather_add(xs, xt, row, col):
    N, H = xs.shape
    E = row.shape[0]
    epw = E // NW
    nch = epw // CH
    ngrp = nch // NBUF
    ng16 = H // 16

    @functools.partial(
        pl.kernel,
        out_type=jax.ShapeDtypeStruct((E, H), jnp.float32),
        mesh=_vmesh(),
        scratch_types=(
            [pltpu.VMEM((epw,), jnp.int32)] * 2
            + [pltpu.VMEM((CH, H), jnp.float32)] * (2 * NBUF)
            + [pltpu.SemaphoreType.DMA] * (2 * NBUF)
        ),
    )
    def k(xs_hbm, xt_hbm, row_hbm, col_hbm, pre_hbm, *scr):
        idx_r, idx_c = scr[0], scr[1]
        rs = scr[2:2 + NBUF]
        rt = scr[2 + NBUF:2 + 2 * NBUF]
        sem_g = scr[2 + 2 * NBUF:2 + 3 * NBUF]
        sem_w = scr[2 + 3 * NBUF:2 + 4 * NBUF]
        wid = lax.axis_index("s") * NC + lax.axis_index("c")
        base = wid * epw
        pltpu.sync_copy(row_hbm.at[pl.ds(base, epw)], idx_r)
        pltpu.sync_copy(col_hbm.at[pl.ds(base, epw)], idx_c)

        def gstart(b, off):
            pltpu.make_async_copy(
                xs_hbm.at[idx_r.at[pl.ds(off, CH)]], rs[b], sem_g[b]).start()
            pltpu.make_async_copy(
                xt_hbm.at[idx_c.at[pl.ds(off, CH)]], rt[b], sem_g[b]).start()

        def gwait(b, off):
            pltpu.make_async_copy(
                xs_hbm.at[idx_r.at[pl.ds(off, CH)]], rs[b], sem_g[b]).wait()
            pltpu.make_async_copy(
                xt_hbm.at[idx_c.at[pl.ds(off, CH)]], rt[b], sem_g[b]).wait()

        def add_rows(b):
            def rowstep(i, carry):
                for j in range(ng16):
                    sl = pl.ds(j * 16, 16)
                    rs[b][i, sl] = rs[b][i, sl] + rt[b][i, sl]
                return carry
            lax.fori_loop(0, CH, rowstep, 0)

        def wstart(b, off):
            pltpu.make_async_copy(
                rs[b], pre_hbm.at[pl.ds(base + off, CH)], sem_w[b]).start()

        def wwait(b, off):
            pltpu.make_async_copy(
                rs[b], pre_hbm.at[pl.ds(base + off, CH)], sem_w[b]).wait()

        for b in range(NBUF):
            gstart(b, b * CH)

        def group(g, carry):
            for b in range(NBUF):
                off = (g * NBUF + b) * CH
                gwait(b, off)
                add_rows(b)
                wstart(b, off)
                wwait(b, off)
                gstart(b, off + NBUF * CH)
            return carry

        lax.fori_loop(0, ngrp - 1, group, 0)
        for b in range(NBUF):
            off = ((ngrp - 1) * NBUF + b) * CH
            gwait(b, off)
            add_rows(b)
            wstart(b, off)
            wwait(b, off)

    return k(xs, xt, row, col)


def _sc_scatter(ef, row, zrows, Np):
    E, H = ef.shape
    CHS = 40
    epw = E // NW
    nch = epw // CHS
    rpt = Np // NS
    zb = zrows.shape[0]

    ngrp = nch // NBUF

    @functools.partial(
        pl.kernel,
        out_type=jax.ShapeDtypeStruct((NC, Np, H), jnp.float32),
        mesh=_vmesh(),
        scratch_types=(
            [pltpu.VMEM_SHARED((Np, H), jnp.float32)]
            + [pltpu.VMEM((CHS, H), jnp.float32)] * NBUF
            + [pltpu.VMEM((CHS,), jnp.int32)] * NBUF
            + [pltpu.VMEM((zb, H), jnp.float32)]
            + [pltpu.SemaphoreType.DMA] * (2 * NBUF + 1)
        ),
    )
    def k(ef_hbm, row_hbm, z_hbm, out_hbm, *scr):
        acc = scr[0]
        efb = scr[1:1 + NBUF]
        idxb = scr[1 + NBUF:1 + 2 * NBUF]
        zbuf = scr[1 + 2 * NBUF]
        sem_l = scr[2 + 2 * NBUF:2 + 3 * NBUF]
        sem_a = scr[2 + 3 * NBUF:2 + 4 * NBUF]
        sem_z = scr[2 + 4 * NBUF]
        cid = lax.axis_index("c")
        sid = lax.axis_index("s")
        wid = sid * NC + cid
        base = wid * epw
        pltpu.sync_copy(z_hbm, zbuf)
        for j in range(rpt // zb):
            pltpu.make_async_copy(
                zbuf, acc.at[pl.ds(sid * rpt + j * zb, zb)], sem_z).start()
        for j in range(rpt // zb):
            pltpu.make_async_copy(
                zbuf, acc.at[pl.ds(sid * rpt + j * zb, zb)], sem_z).wait()
        plsc.subcore_barrier()

        def lstart(b, off):
            pltpu.make_async_copy(
                row_hbm.at[pl.ds(base + off, CHS)], idxb[b], sem_l[b]).start()
            pltpu.make_async_copy(
                ef_hbm.at[pl.ds(base + off, CHS)], efb[b], sem_l[b]).start()

        def lwait(b, off):
            pltpu.make_async_copy(
                row_hbm.at[pl.ds(base + off, CHS)], idxb[b], sem_l[b]).wait()
            pltpu.make_async_copy(
                ef_hbm.at[pl.ds(base + off, CHS)], efb[b], sem_l[b]).wait()

        for b in range(NBUF):
            lstart(b, b * CHS)

        def group(g, carry):
            for b in range(NBUF):
                off = (g * NBUF + b) * CHS
                lwait(b, off)
                pltpu.sync_copy(efb[b], acc.at[idxb[b]], add=True)
                lstart(b, off + NBUF * CHS)
            return carry

        lax.fori_loop(0, ngrp - 1, group, 0)
        for b in range(NBUF):
            off = ((ngrp - 1) * NBUF + b) * CHS
            lwait(b, off)
            pltpu.sync_copy(efb[b], acc.at[idxb[b]], add=True)
        plsc.subcore_barrier()

        nfl = rpt // CHS
        for j in range(nfl):
            b = j % NBUF
            r0 = sid * rpt + j * CHS
            if j >= NBUF:
                pltpu.make_async_copy(
                    efb[b], out_hbm.at[cid, pl.ds(sid * rpt + (j - NBUF) * CHS, CHS)],
                    sem_a[b]).wait()
            pltpu.sync_copy(acc.at[pl.ds(r0, CHS)], efb[b])
            pltpu.make_async_copy(
                efb[b], out_hbm.at[cid, pl.ds(r0, CHS)], sem_a[b]).start()
        for j in range(max(nfl - NBUF, 0), nfl):
            b = j % NBUF
            pltpu.make_async_copy(
                efb[b], out_hbm.at[cid, pl.ds(sid * rpt + j * CHS, CHS)],
                sem_a[b]).wait()

    return k(ef, row, zrows)


def kernel(x, edge_index, edge_mask, edge_attr,
           We1, be1, We2, be2, Wn1, bn1, Wn2, bn2):
    N, D = x.shape
    E = edge_index.shape[1]
    H = We2.shape[0]
    Do = Wn2.shape[1]
    row = edge_index[0]
    col = edge_index[1]
    Ws, Wt, Wa = We1[:D], We1[D:2 * D], We1[2 * D:]

    xs, xt = _tc_precompute(x, Ws, Wt, bn=1000)
    pre = _sc_gather_add(xs, xt, row, col)
    ef = _tc_edge(pre, edge_attr, edge_mask, Wa,
                  be1.reshape(1, H), We2, be2.reshape(1, H), be=2000)
    Np = ((N + 128 * NS - 1) // (128 * NS)) * (128 * NS)
    z = jnp.zeros((128, H), jnp.float32)
    parts = _sc_scatter(ef, row, z, Np)
    x_out = _tc_node(x, parts[0], parts[1], Wn1[:D], Wn1[D:],
                     bn1.reshape(1, H), Wn2, bn2.reshape(1, Do), bn=1000)
    x_out, *_ = lax.optimization_barrier(
        (x_out, row, col, xs, xt, pre, z, parts))
    return (x_out, ef)


# speedup vs baseline: 1.2589x; 1.0685x over previous
import functools

import jax
import jax.numpy as jnp
from jax import lax
from jax.experimental import pallas as pl
from jax.experimental.pallas import tpu as pltpu
from jax.experimental.pallas import tpu_sc as plsc

NC = 2
NS = 16
NW = NC * NS
CH = 80


def _vmesh():
    return plsc.VectorSubcoreMesh(
        core_axis_name="c", subcore_axis_name="s",
        num_cores=NC, num_subcores=NS)


def _pre_body(x_ref, ws_ref, wt_ref, xs_ref, xt_ref):
    x = x_ref[...]
    xs_ref[...] = jnp.dot(x, ws_ref[...], preferred_element_type=jnp.float32)
    xt_ref[...] = jnp.dot(x, wt_ref[...], preferred_element_type=jnp.float32)


def _tc_precompute(x, Ws, Wt, bn):
    N, D = x.shape
    H = Ws.shape[1]
    grid = (N // bn,)
    return pl.pallas_call(
        _pre_body,
        grid=grid,
        in_specs=[pl.BlockSpec((bn, D), lambda i: (i, 0)),
                  pl.BlockSpec((D, H), lambda i: (0, 0)),
                  pl.BlockSpec((D, H), lambda i: (0, 0))],
        out_specs=[pl.BlockSpec((bn, H), lambda i: (i, 0)),
                   pl.BlockSpec((bn, H), lambda i: (i, 0))],
        out_shape=[jax.ShapeDtypeStruct((N, H), jnp.float32),
                   jax.ShapeDtypeStruct((N, H), jnp.float32)],
    )(x, Ws, Wt)


def _edge_body(pre_ref, attr_ref, mask_ref, wa_ref, be1_ref,
               we2_ref, be2_ref, ef_ref):
    pre = (pre_ref[...]
           + jnp.dot(attr_ref[...], wa_ref[...],
                     preferred_element_type=jnp.float32)
           + be1_ref[...])
    h = jnp.maximum(pre, 0.0)
    ef_ref[...] = (jnp.dot(h, we2_ref[...],
                           preferred_element_type=jnp.float32)
                   + be2_ref[...]) * mask_ref[...]


def _tc_edge(pre, attr, mask, Wa, be1, We2, be2, be):
    E, H = pre.shape
    DE = attr.shape[1]
    grid = (E // be,)
    return pl.pallas_call(
        _edge_body,
        grid=grid,
        in_specs=[pl.BlockSpec((be, H), lambda i: (i, 0)),
                  pl.BlockSpec((be, DE), lambda i: (i, 0)),
                  pl.BlockSpec((be, 1), lambda i: (i, 0)),
                  pl.BlockSpec((DE, H), lambda i: (0, 0)),
                  pl.BlockSpec((1, H), lambda i: (0, 0)),
                  pl.BlockSpec((H, H), lambda i: (0, 0)),
                  pl.BlockSpec((1, H), lambda i: (0, 0))],
        out_specs=pl.BlockSpec((be, H), lambda i: (i, 0)),
        out_shape=jax.ShapeDtypeStruct((E, H), jnp.float32),
    )(pre, attr, mask, Wa, be1, We2, be2)


def _node_body(x_ref, p0_ref, p1_ref, w1x_ref, w1a_ref, bn1_ref,
               w2_ref, bn2_ref, o_ref):
    agg = p0_ref[...] + p1_ref[...]
    h = jnp.maximum(
        jnp.dot(x_ref[...], w1x_ref[...], preferred_element_type=jnp.float32)
        + jnp.dot(agg, w1a_ref[...], preferred_element_type=jnp.float32)
        + bn1_ref[...], 0.0)
    o_ref[...] = (jnp.dot(h, w2_ref[...], preferred_element_type=jnp.float32)
                  + bn2_ref[...])


def _tc_node(x, p0, p1, W1x, W1a, bn1, W2, bn2, bn):
    N, D = x.shape
    H = W1x.shape[1]
    Do = W2.shape[1]
    grid = (N // bn,)
    return pl.pallas_call(
        _node_body,
        grid=grid,
        in_specs=[pl.BlockSpec((bn, D), lambda i: (i, 0)),
                  pl.BlockSpec((bn, H), lambda i: (i, 0)),
                  pl.BlockSpec((bn, H), lambda i: (i, 0)),
                  pl.BlockSpec((D, H), lambda i: (0, 0)),
                  pl.BlockSpec((H, H), lambda i: (0, 0)),
                  pl.BlockSpec((1, H), lambda i: (0, 0)),
                  pl.BlockSpec((H, Do), lambda i: (0, 0)),
                  pl.BlockSpec((1, Do), lambda i: (0, 0))],
        out_specs=pl.BlockSpec((bn, Do), lambda i: (i, 0)),
        out_shape=jax.ShapeDtypeStruct((N, Do), jnp.float32),
    )(x, p0, p1, W1x, W1a, bn1, W2, bn2)


NBUF = 5


def _sc_gather_add(xs, xt, row, col):
    N, H = xs.shape
    E = row.shape[0]
    epw = E // NW
    nch = epw // CH
    ngrp = nch // NBUF
    ng16 = H // 16

    @functools.partial(
        pl.kernel,
        out_type=jax.ShapeDtypeStruct((E, H), jnp.float32),
        mesh=_vmesh(),
        scratch_types=(
            [pltpu.VMEM((epw,), jnp.int32)] * 2
            + [pltpu.VMEM((CH, H), jnp.float32)] * (2 * NBUF)
            + [pltpu.SemaphoreType.DMA] * (2 * NBUF)
        ),
    )
    def k(xs_hbm, xt_hbm, row_hbm, col_hbm, pre_hbm, *scr):
        idx_r, idx_c = scr[0], scr[1]
        rs = scr[2:2 + NBUF]
        rt = scr[2 + NBUF:2 + 2 * NBUF]
        sem_g = scr[2 + 2 * NBUF:2 + 3 * NBUF]
        sem_w = scr[2 + 3 * NBUF:2 + 4 * NBUF]
        wid = lax.axis_index("s") * NC + lax.axis_index("c")
        base = wid * epw
        pltpu.sync_copy(row_hbm.at[pl.ds(base, epw)], idx_r)
        pltpu.sync_copy(col_hbm.at[pl.ds(base, epw)], idx_c)

        def gstart(b, off):
            pltpu.make_async_copy(
                xs_hbm.at[idx_r.at[pl.ds(off, CH)]], rs[b], sem_g[b]).start()
            pltpu.make_async_copy(
                xt_hbm.at[idx_c.at[pl.ds(off, CH)]], rt[b], sem_g[b]).start()

        def gwait(b, off):
            pltpu.make_async_copy(
                xs_hbm.at[idx_r.at[pl.ds(off, CH)]], rs[b], sem_g[b]).wait()
            pltpu.make_async_copy(
                xt_hbm.at[idx_c.at[pl.ds(off, CH)]], rt[b], sem_g[b]).wait()

        def add_rows(b):
            def rowstep(i, carry):
                for j in range(ng16):
                    sl = pl.ds(j * 16, 16)
                    rs[b][i, sl] = rs[b][i, sl] + rt[b][i, sl]
                return carry
            lax.fori_loop(0, CH, rowstep, 0)

        def wstart(b, off):
            pltpu.make_async_copy(
                rs[b], pre_hbm.at[pl.ds(base + off, CH)], sem_w[b]).start()

        def wwait(b, off):
            pltpu.make_async_copy(
                rs[b], pre_hbm.at[pl.ds(base + off, CH)], sem_w[b]).wait()

        for b in range(NBUF):
            gstart(b, b * CH)

        def group(g, carry):
            for b in range(NBUF):
                off = (g * NBUF + b) * CH
                gwait(b, off)
                add_rows(b)
                wstart(b, off)
                wwait(b, off)
                gstart(b, off + NBUF * CH)
            return carry

        lax.fori_loop(0, ngrp - 1, group, 0)
        for b in range(NBUF):
            off = ((ngrp - 1) * NBUF + b) * CH
            gwait(b, off)
            add_rows(b)
            wstart(b, off)
            wwait(b, off)

    return k(xs, xt, row, col)


def _sc_scatter(ef, row, zrows, Np):
    E, H = ef.shape
    CHS = 40
    epw = E // NW
    nch = epw // CHS
    rpt = Np // NS
    zb = zrows.shape[0]

    ngrp = nch // NBUF

    @functools.partial(
        pl.kernel,
        out_type=jax.ShapeDtypeStruct((NC, Np, H), jnp.float32),
        mesh=_vmesh(),
        scratch_types=(
            [pltpu.VMEM_SHARED((Np, H), jnp.float32)]
            + [pltpu.VMEM((CHS, H), jnp.float32)] * NBUF
            + [pltpu.VMEM((CHS,), jnp.int32)] * NBUF
            + [pltpu.VMEM((zb, H), jnp.float32)]
            + [pltpu.SemaphoreType.DMA] * (2 * NBUF + 1)
        ),
    )
    def k(ef_hbm, row_hbm, z_hbm, out_hbm, *scr):
        acc = scr[0]
        efb = scr[1:1 + NBUF]
        idxb = scr[1 + NBUF:1 + 2 * NBUF]
        zbuf = scr[1 + 2 * NBUF]
        sem_l = scr[2 + 2 * NBUF:2 + 3 * NBUF]
        sem_a = scr[2 + 3 * NBUF:2 + 4 * NBUF]
        sem_z = scr[2 + 4 * NBUF]
        cid = lax.axis_index("c")
        sid = lax.axis_index("s")
        wid = sid * NC + cid
        base = wid * epw
        pltpu.sync_copy(z_hbm, zbuf)
        for j in range(rpt // zb):
            pltpu.make_async_copy(
                zbuf, acc.at[pl.ds(sid * rpt + j * zb, zb)], sem_z).start()
        for j in range(rpt // zb):
            pltpu.make_async_copy(
                zbuf, acc.at[pl.ds(sid * rpt + j * zb, zb)], sem_z).wait()
        plsc.subcore_barrier()

        def lstart(b, off):
            pltpu.make_async_copy(
                row_hbm.at[pl.ds(base + off, CHS)], idxb[b], sem_l[b]).start()
            pltpu.make_async_copy(
                ef_hbm.at[pl.ds(base + off, CHS)], efb[b], sem_l[b]).start()

        def lwait(b, off):
            pltpu.make_async_copy(
                row_hbm.at[pl.ds(base + off, CHS)], idxb[b], sem_l[b]).wait()
            pltpu.make_async_copy(
                ef_hbm.at[pl.ds(base + off, CHS)], efb[b], sem_l[b]).wait()

        for b in range(NBUF):
            lstart(b, b * CHS)

        def group(g, carry):
            for b in range(NBUF):
                off = (g * NBUF + b) * CHS
                lwait(b, off)
                pltpu.sync_copy(efb[b], acc.at[idxb[b]], add=True)
                lstart(b, off + NBUF * CHS)
            return carry

        lax.fori_loop(0, ngrp - 1, group, 0)
        for b in range(NBUF):
            off = ((ngrp - 1) * NBUF + b) * CHS
            lwait(b, off)
            pltpu.sync_copy(efb[b], acc.at[idxb[b]], add=True)
        plsc.subcore_barrier()

        nfl = rpt // CHS
        for j in range(nfl):
            b = j % NBUF
            r0 = sid * rpt + j * CHS
            if j >= NBUF:
                pltpu.make_async_copy(
                    efb[b], out_hbm.at[cid, pl.ds(sid * rpt + (j - NBUF) * CHS, CHS)],
                    sem_a[b]).wait()
            pltpu.sync_copy(acc.at[pl.ds(r0, CHS)], efb[b])
            pltpu.make_async_copy(
                efb[b], out_hbm.at[cid, pl.ds(r0, CHS)], sem_a[b]).start()
        for j in range(max(nfl - NBUF, 0), nfl):
            b = j % NBUF
            pltpu.make_async_copy(
                efb[b], out_hbm.at[cid, pl.ds(sid * rpt + j * CHS, CHS)],
                sem_a[b]).wait()

    return k(ef, row, zrows)


def kernel(x, edge_index, edge_mask, edge_attr,
           We1, be1, We2, be2, Wn1, bn1, Wn2, bn2):
    N, D = x.shape
    E = edge_index.shape[1]
    H = We2.shape[0]
    Do = Wn2.shape[1]
    row = edge_index[0]
    col = edge_index[1]
    Ws, Wt, Wa = We1[:D], We1[D:2 * D], We1[2 * D:]

    xs, xt = _tc_precompute(x, Ws, Wt, bn=2000)
    pre = _sc_gather_add(xs, xt, row, col)
    ef = _tc_edge(pre, edge_attr, edge_mask, Wa,
                  be1.reshape(1, H), We2, be2.reshape(1, H), be=4000)
    Np = ((N + 128 * NS - 1) // (128 * NS)) * (128 * NS)
    z = jnp.zeros((128, H), jnp.float32)
    parts = _sc_scatter(ef, row, z, Np)
    x_out = _tc_node(x, parts[0], parts[1], Wn1[:D], Wn1[D:],
                     bn1.reshape(1, H), Wn2, bn2.reshape(1, Do), bn=2000)
    x_out, *_ = lax.optimization_barrier(
        (x_out, row, col, xs, xt, pre, z, parts))
    return (x_out, ef)


# speedup vs baseline: 1.2675x; 1.0068x over previous
import functools

import jax
import jax.numpy as jnp
from jax import lax
from jax.experimental import pallas as pl
from jax.experimental.pallas import tpu as pltpu
from jax.experimental.pallas import tpu_sc as plsc

NC = 2
NS = 16
NW = NC * NS
CH = 80


def _vmesh():
    return plsc.VectorSubcoreMesh(
        core_axis_name="c", subcore_axis_name="s",
        num_cores=NC, num_subcores=NS)


def _pre_body(x_ref, ws_ref, wt_ref, xs_ref, xt_ref):
    x = x_ref[...]
    xs_ref[...] = jnp.dot(x, ws_ref[...], preferred_element_type=jnp.float32)
    xt_ref[...] = jnp.dot(x, wt_ref[...], preferred_element_type=jnp.float32)


def _tc_precompute(x, Ws, Wt, bn):
    N, D = x.shape
    H = Ws.shape[1]
    grid = (N // bn,)
    return pl.pallas_call(
        _pre_body,
        grid=grid,
        in_specs=[pl.BlockSpec((bn, D), lambda i: (i, 0)),
                  pl.BlockSpec((D, H), lambda i: (0, 0)),
                  pl.BlockSpec((D, H), lambda i: (0, 0))],
        out_specs=[pl.BlockSpec((bn, H), lambda i: (i, 0)),
                   pl.BlockSpec((bn, H), lambda i: (i, 0))],
        out_shape=[jax.ShapeDtypeStruct((N, H), jnp.float32),
                   jax.ShapeDtypeStruct((N, H), jnp.float32)],
    )(x, Ws, Wt)


def _edge_body(pre_ref, attr_ref, mask_ref, wa_ref, be1_ref,
               we2_ref, be2_ref, ef_ref):
    pre = (pre_ref[...]
           + jnp.dot(attr_ref[...], wa_ref[...],
                     preferred_element_type=jnp.float32)
           + be1_ref[...])
    h = jnp.maximum(pre, 0.0)
    ef_ref[...] = (jnp.dot(h, we2_ref[...],
                           preferred_element_type=jnp.float32)
                   + be2_ref[...]) * mask_ref[...]


def _tc_edge(pre, attr, mask, Wa, be1, We2, be2, be):
    E, H = pre.shape
    DE = attr.shape[1]
    grid = (E // be,)
    return pl.pallas_call(
        _edge_body,
        grid=grid,
        in_specs=[pl.BlockSpec((be, H), lambda i: (i, 0)),
                  pl.BlockSpec((be, DE), lambda i: (i, 0)),
                  pl.BlockSpec((be, 1), lambda i: (i, 0)),
                  pl.BlockSpec((DE, H), lambda i: (0, 0)),
                  pl.BlockSpec((1, H), lambda i: (0, 0)),
                  pl.BlockSpec((H, H), lambda i: (0, 0)),
                  pl.BlockSpec((1, H), lambda i: (0, 0))],
        out_specs=pl.BlockSpec((be, H), lambda i: (i, 0)),
        out_shape=jax.ShapeDtypeStruct((E, H), jnp.float32),
    )(pre, attr, mask, Wa, be1, We2, be2)


def _node_body(x_ref, p0_ref, p1_ref, w1x_ref, w1a_ref, bn1_ref,
               w2_ref, bn2_ref, o_ref):
    agg = p0_ref[...] + p1_ref[...]
    h = jnp.maximum(
        jnp.dot(x_ref[...], w1x_ref[...], preferred_element_type=jnp.float32)
        + jnp.dot(agg, w1a_ref[...], preferred_element_type=jnp.float32)
        + bn1_ref[...], 0.0)
    o_ref[...] = (jnp.dot(h, w2_ref[...], preferred_element_type=jnp.float32)
                  + bn2_ref[...])


def _tc_node(x, p0, p1, W1x, W1a, bn1, W2, bn2, bn):
    N, D = x.shape
    H = W1x.shape[1]
    Do = W2.shape[1]
    grid = (N // bn,)
    return pl.pallas_call(
        _node_body,
        grid=grid,
        in_specs=[pl.BlockSpec((bn, D), lambda i: (i, 0)),
                  pl.BlockSpec((bn, H), lambda i: (i, 0)),
                  pl.BlockSpec((bn, H), lambda i: (i, 0)),
                  pl.BlockSpec((D, H), lambda i: (0, 0)),
                  pl.BlockSpec((H, H), lambda i: (0, 0)),
                  pl.BlockSpec((1, H), lambda i: (0, 0)),
                  pl.BlockSpec((H, Do), lambda i: (0, 0)),
                  pl.BlockSpec((1, Do), lambda i: (0, 0))],
        out_specs=pl.BlockSpec((bn, Do), lambda i: (i, 0)),
        out_shape=jax.ShapeDtypeStruct((N, Do), jnp.float32),
    )(x, p0, p1, W1x, W1a, bn1, W2, bn2)


NBUF = 5


def _sc_gather_add(xs, xt, row, col):
    N, H = xs.shape
    E = row.shape[0]
    epw = E // NW
    nch = epw // CH
    ngrp = nch // NBUF
    ng16 = H // 16

    @functools.partial(
        pl.kernel,
        out_type=jax.ShapeDtypeStruct((E, H), jnp.float32),
        mesh=_vmesh(),
        scratch_types=(
            [pltpu.VMEM((epw,), jnp.int32)] * 2
            + [pltpu.VMEM((CH, H), jnp.float32)] * (2 * NBUF)
            + [pltpu.SemaphoreType.DMA] * (2 * NBUF)
        ),
    )
    def k(xs_hbm, xt_hbm, row_hbm, col_hbm, pre_hbm, *scr):
        idx_r, idx_c = scr[0], scr[1]
        rs = scr[2:2 + NBUF]
        rt = scr[2 + NBUF:2 + 2 * NBUF]
        sem_g = scr[2 + 2 * NBUF:2 + 3 * NBUF]
        sem_w = scr[2 + 3 * NBUF:2 + 4 * NBUF]
        wid = lax.axis_index("s") * NC + lax.axis_index("c")
        base = wid * epw
        pltpu.sync_copy(row_hbm.at[pl.ds(base, epw)], idx_r)
        pltpu.sync_copy(col_hbm.at[pl.ds(base, epw)], idx_c)

        def gstart(b, off):
            pltpu.make_async_copy(
                xs_hbm.at[idx_r.at[pl.ds(off, CH)]], rs[b], sem_g[b]).start()
            pltpu.make_async_copy(
                xt_hbm.at[idx_c.at[pl.ds(off, CH)]], rt[b], sem_g[b]).start()

        def gwait(b, off):
            pltpu.make_async_copy(
                xs_hbm.at[idx_r.at[pl.ds(off, CH)]], rs[b], sem_g[b]).wait()
            pltpu.make_async_copy(
                xt_hbm.at[idx_c.at[pl.ds(off, CH)]], rt[b], sem_g[b]).wait()

        def add_rows(b):
            def rowstep(i, carry):
                for j in range(ng16):
                    sl = pl.ds(j * 16, 16)
                    rs[b][i, sl] = rs[b][i, sl] + rt[b][i, sl]
                return carry
            lax.fori_loop(0, CH, rowstep, 0)

        def wstart(b, off):
            pltpu.make_async_copy(
                rs[b], pre_hbm.at[pl.ds(base + off, CH)], sem_w[b]).start()

        def wwait(b, off):
            pltpu.make_async_copy(
                rs[b], pre_hbm.at[pl.ds(base + off, CH)], sem_w[b]).wait()

        for b in range(NBUF):
            gstart(b, b * CH)

        def group(g, carry):
            for b in range(NBUF):
                off = (g * NBUF + b) * CH
                gwait(b, off)
                add_rows(b)
                wstart(b, off)
                wwait(b, off)
                gstart(b, off + NBUF * CH)
            return carry

        lax.fori_loop(0, ngrp - 1, group, 0)
        for b in range(NBUF):
            off = ((ngrp - 1) * NBUF + b) * CH
            gwait(b, off)
            add_rows(b)
            wstart(b, off)
            wwait(b, off)

    return k(xs, xt, row, col)


def _sc_scatter(ef, row, zrows, Np):
    E, H = ef.shape
    CHS = 40
    epw = E // NW
    nch = epw // CHS
    rpt = Np // NS
    zb = zrows.shape[0]

    ngrp = nch // NBUF

    @functools.partial(
        pl.kernel,
        out_type=jax.ShapeDtypeStruct((NC, Np, H), jnp.float32),
        mesh=_vmesh(),
        scratch_types=(
            [pltpu.VMEM_SHARED((Np, H), jnp.float32)]
            + [pltpu.VMEM((CHS, H), jnp.float32)] * NBUF
            + [pltpu.VMEM((CHS,), jnp.int32)] * NBUF
            + [pltpu.VMEM((zb, H), jnp.float32)]
            + [pltpu.SemaphoreType.DMA] * (2 * NBUF + 1)
        ),
    )
    def k(ef_hbm, row_hbm, z_hbm, out_hbm, *scr):
        acc = scr[0]
        efb = scr[1:1 + NBUF]
        idxb = scr[1 + NBUF:1 + 2 * NBUF]
        zbuf = scr[1 + 2 * NBUF]
        sem_l = scr[2 + 2 * NBUF:2 + 3 * NBUF]
        sem_a = scr[2 + 3 * NBUF:2 + 4 * NBUF]
        sem_z = scr[2 + 4 * NBUF]
        cid = lax.axis_index("c")
        sid = lax.axis_index("s")
        wid = sid * NC + cid
        base = wid * epw
        pltpu.sync_copy(z_hbm, zbuf)
        for j in range(rpt // zb):
            pltpu.make_async_copy(
                zbuf, acc.at[pl.ds(sid * rpt + j * zb, zb)], sem_z).start()
        for j in range(rpt // zb):
            pltpu.make_async_copy(
                zbuf, acc.at[pl.ds(sid * rpt + j * zb, zb)], sem_z).wait()
        plsc.subcore_barrier()

        def lstart(b, off):
            pltpu.make_async_copy(
                row_hbm.at[pl.ds(base + off, CHS)], idxb[b], sem_l[b]).start()
            pltpu.make_async_copy(
                ef_hbm.at[pl.ds(base + off, CHS)], efb[b], sem_l[b]).start()

        def lwait(b, off):
            pltpu.make_async_copy(
                row_hbm.at[pl.ds(base + off, CHS)], idxb[b], sem_l[b]).wait()
            pltpu.make_async_copy(
                ef_hbm.at[pl.ds(base + off, CHS)], efb[b], sem_l[b]).wait()

        for b in range(NBUF):
            lstart(b, b * CHS)

        def group(g, carry):
            for b in range(NBUF):
                off = (g * NBUF + b) * CHS
                lwait(b, off)
                pltpu.sync_copy(efb[b], acc.at[idxb[b]], add=True)
                lstart(b, off + NBUF * CHS)
            return carry

        lax.fori_loop(0, ngrp - 1, group, 0)
        for b in range(NBUF):
            off = ((ngrp - 1) * NBUF + b) * CHS
            lwait(b, off)
            pltpu.sync_copy(efb[b], acc.at[idxb[b]], add=True)
        plsc.subcore_barrier()

        nfl = rpt // CHS
        for j in range(nfl):
            b = j % NBUF
            r0 = sid * rpt + j * CHS
            if j >= NBUF:
                pltpu.make_async_copy(
                    efb[b], out_hbm.at[cid, pl.ds(sid * rpt + (j - NBUF) * CHS, CHS)],
                    sem_a[b]).wait()
            pltpu.sync_copy(acc.at[pl.ds(r0, CHS)], efb[b])
            pltpu.make_async_copy(
                efb[b], out_hbm.at[cid, pl.ds(r0, CHS)], sem_a[b]).start()
        for j in range(max(nfl - NBUF, 0), nfl):
            b = j % NBUF
            pltpu.make_async_copy(
                efb[b], out_hbm.at[cid, pl.ds(sid * rpt + j * CHS, CHS)],
                sem_a[b]).wait()

    return k(ef, row, zrows)


def kernel(x, edge_index, edge_mask, edge_attr,
           We1, be1, We2, be2, Wn1, bn1, Wn2, bn2):
    N, D = x.shape
    E = edge_index.shape[1]
    H = We2.shape[0]
    Do = Wn2.shape[1]
    row = edge_index[0]
    col = edge_index[1]
    Ws, Wt, Wa = We1[:D], We1[D:2 * D], We1[2 * D:]

    xs, xt = _tc_precompute(x, Ws, Wt, bn=2000)
    pre = _sc_gather_add(xs, xt, row, col)
    ef = _tc_edge(pre, edge_attr, edge_mask, Wa,
                  be1.reshape(1, H), We2, be2.reshape(1, H), be=8000)
    Np = ((N + 128 * NS - 1) // (128 * NS)) * (128 * NS)
    z = jnp.zeros((128, H), jnp.float32)
    parts = _sc_scatter(ef, row, z, Np)
    x_out = _tc_node(x, parts[0], parts[1], Wn1[:D], Wn1[D:],
                     bn1.reshape(1, H), Wn2, bn2.reshape(1, Do), bn=2000)
    x_out, *_ = lax.optimization_barrier(
        (x_out, row, col, xs, xt, pre, z, parts))
    return (x_out, ef)
